# baseline (speedup 1.0000x reference)
import functools

import jax
import jax.numpy as jnp
import numpy as np
from jax.experimental import pallas as pl

_TINY = np.float32(np.finfo(np.float32).tiny)
_NEG_INF = np.float32(-np.inf)
_INT_MAX = np.int32(np.iinfo(np.int32).max)


def _gumbel_from_counter(t):
    ks1 = jnp.uint32(42)
    ks2 = jnp.uint32(0 ^ 42 ^ 0x1BD11BDA)
    ks0 = jnp.uint32(0)
    inj = ((ks1, ks2 + jnp.uint32(1)), (ks2, ks0 + jnp.uint32(2)),
           (ks0, ks1 + jnp.uint32(3)), (ks1, ks2 + jnp.uint32(4)),
           (ks2, ks0 + jnp.uint32(5)))
    rots = ((13, 15, 26, 6), (17, 29, 16, 24))
    x0 = t
    x1 = ((t << jnp.uint32(13)) | (t >> jnp.uint32(19))) ^ t
    first = True
    for g in range(5):
        for r in rots[g & 1]:
            if first:
                first = False
                continue
            x0 = x0 + x1
            x1 = ((x1 << jnp.uint32(r)) | (x1 >> jnp.uint32(32 - r))) ^ x0
        a, b = inj[g]
        x0 = x0 + a
        x1 = x1 + b
    bits = x0 ^ x1
    fb = (bits >> jnp.uint32(9)) | jnp.uint32(0x3F800000)
    f = jax.lax.bitcast_convert_type(fb, jnp.float32) - jnp.float32(1.0)
    u = jnp.maximum(_TINY, f)
    return -jnp.log(-jnp.log(u))


def _fold_body(x_ref, base_ref, hi_ref, acc_ref, blk_ref, *, vb):
    j = pl.program_id(0)
    t = base_ref[...] + (j * vb).astype(jnp.uint32)
    s = _gumbel_from_counter(t) + x_ref[...]
    s = jnp.where(t < hi_ref[...], s, _NEG_INF)
    a = jnp.where(j == 0, _NEG_INF, acc_ref[...])
    acc_ref[...] = jnp.maximum(a, s)
    blk_ref[...] = jnp.where(s > a, j, blk_ref[...])


def _merge_body(acc_ref, blk_ref, o_ref, *, vb):
    a = acc_ref[...]
    col = jax.lax.broadcasted_iota(jnp.int32, a.shape, 1)
    gidx = blk_ref[...] * vb + col
    rowmax = jnp.max(a, axis=1, keepdims=True)
    cand = jnp.where(a == rowmax, gidx, _INT_MAX)
    o_ref[...] = jnp.min(cand, axis=1, keepdims=True)


@jax.jit
def kernel(inputs):
    b, nv = inputs.shape
    vb = 2048
    nb = pl.cdiv(nv, vb)
    rows = jnp.arange(b, dtype=jnp.int32) * nv
    cols = jnp.arange(vb, dtype=jnp.int32)
    base = (rows[:, None] + cols[None, :] + 42).astype(jnp.uint32)
    hi = jnp.broadcast_to((rows[:, None] + (nv + 42)).astype(jnp.uint32),
                          (b, vb))
    acc, blk = pl.pallas_call(
        functools.partial(_fold_body, vb=vb),
        grid=(nb,),
        in_specs=[pl.BlockSpec((b, vb), lambda j: (0, j)),
                  pl.BlockSpec((b, vb), lambda j: (0, 0)),
                  pl.BlockSpec((b, vb), lambda j: (0, 0))],
        out_specs=[pl.BlockSpec((b, vb), lambda j: (0, 0)),
                   pl.BlockSpec((b, vb), lambda j: (0, 0))],
        out_shape=[jax.ShapeDtypeStruct((b, vb), jnp.float32),
                   jax.ShapeDtypeStruct((b, vb), jnp.int32)],
    )(inputs, base, hi)
    out = pl.pallas_call(
        functools.partial(_merge_body, vb=vb),
        in_specs=[pl.BlockSpec((b, vb), lambda: (0, 0)),
                  pl.BlockSpec((b, vb), lambda: (0, 0))],
        out_specs=pl.BlockSpec((b, 1), lambda: (0, 0)),
        out_shape=jax.ShapeDtypeStruct((b, 1), jnp.int32),
    )(acc, blk)
    return out.reshape(b)
